# baseline (speedup 1.0000x reference)
import functools

import jax
import jax.numpy as jnp
from jax import lax
from jax.experimental import pallas as pl
from jax.experimental.pallas import tpu as pltpu
from jax.experimental.pallas import tpu_sc as plsc

_N = 100000
_D = 16
_E = 3200000
_NC = 2
_NS = 16
_NW = _NC * _NS
_LANES = 16
_CHUNK = 1024
_ROWS = _CHUNK // 128
_NCHUNKS = _E // _CHUNK
_K = (_NCHUNKS + _NW - 1) // _NW
_M = _K // 2


def _sqrt16(x):
    i = plsc.bitcast(x, jnp.int32)
    y = plsc.bitcast(jnp.int32(0x5F3759DF) - lax.shift_right_arithmetic(i, jnp.int32(1)),
                     jnp.float32)
    xh = x * jnp.float32(0.5)
    for _ in range(3):
        y = y * (jnp.float32(1.5) - xh * y * y)
    return x * y


def _sc_body(z_hbm, w_hbm, ai_hbm, aj_hbm, si_hbm, sj_hbm, out_hbm,
             ivm, jvm, zr, wr, accv, gsem, isem):
    wid = lax.axis_index("s") * _NC + lax.axis_index("c")
    lanes = lax.iota(jnp.int32, _LANES)

    def term(i_hbm, j_hbm, eps, is_exp):
        def rowbase(k):
            c = jnp.minimum(wid + k * jnp.int32(_NW), jnp.int32(_NCHUNKS - 1))
            return c * jnp.int32(_CHUNK)

        def fetch_idx(k, slot):
            s = jnp.int32(slot)
            r0 = rowbase(k)
            pltpu.async_copy(i_hbm.at[pl.ds(r0, _CHUNK)], ivm.at[s],
                             isem.at[s])
            pltpu.async_copy(j_hbm.at[pl.ds(r0, _CHUNK)], jvm.at[s],
                             isem.at[s])

        def wait_idx(slot):
            s = jnp.int32(slot)
            pltpu.make_async_copy(i_hbm.at[pl.ds(0, _CHUNK)], ivm.at[s],
                                  isem.at[s]).wait()
            pltpu.make_async_copy(j_hbm.at[pl.ds(0, _CHUNK)], jvm.at[s],
                                  isem.at[s]).wait()

        def fire_gathers(slot):
            s = jnp.int32(slot)
            pltpu.async_copy(z_hbm.at[ivm.at[s]], zr.at[s], gsem.at[s])
            pltpu.async_copy(w_hbm.at[jvm.at[s]], wr.at[s], gsem.at[s])

        def wait_gathers(slot):
            s = jnp.int32(slot)
            pltpu.make_async_copy(z_hbm.at[ivm.at[s]], zr.at[s],
                                  gsem.at[s]).wait()
            pltpu.make_async_copy(w_hbm.at[jvm.at[s]], wr.at[s],
                                  gsem.at[s]).wait()

        def compute(k, slot, acc):
            zs, ws = zr.at[jnp.int32(slot)], wr.at[jnp.int32(slot)]

            def group(g, a):
                ev = g * jnp.int32(_LANES) + lanes
                s = jnp.zeros((_LANES,), jnp.float32)
                for d in range(_D):
                    dv = jnp.full((_LANES,), d, jnp.int32)
                    zi = plsc.load_gather(zs, [ev, dv])
                    wj = plsc.load_gather(ws, [ev, dv])
                    t = zi - wj
                    if eps:
                        t = t + jnp.float32(eps)
                    s = s + t * t
                dist = _sqrt16(s)
                v = jnp.exp(-dist) if is_exp else dist
                return a + v

            part = plsc.parallel_loop(
                jnp.int32(0), jnp.int32(_CHUNK // _LANES), jnp.int32(1),
                unroll=4, carry=jnp.zeros((_LANES,), jnp.float32))(group)
            valid = (wid + k * jnp.int32(_NW)) < jnp.int32(_NCHUNKS)
            return acc + jnp.where(valid, part, jnp.zeros_like(part))

        fetch_idx(jnp.int32(0), 0)
        wait_idx(0)
        fire_gathers(0)
        fetch_idx(jnp.int32(1), 1)

        def iter2(m, acc):
            k0 = m * jnp.int32(2)
            k1 = k0 + jnp.int32(1)
            last = m >= jnp.int32(_M - 1)
            wait_idx(1)
            fire_gathers(1)

            @pl.when(jnp.logical_not(last))
            def _():
                fetch_idx(k0 + jnp.int32(2), 0)

            wait_gathers(0)
            acc = compute(k0, 0, acc)

            @pl.when(jnp.logical_not(last))
            def _():
                wait_idx(0)
                fire_gathers(0)
                fetch_idx(k1 + jnp.int32(2), 1)

            wait_gathers(1)
            acc = compute(k1, 1, acc)
            return acc

        return lax.fori_loop(jnp.int32(0), jnp.int32(_M), iter2,
                             jnp.zeros((_LANES,), jnp.float32))

    acc_a = term(ai_hbm, aj_hbm, 0.0, True)
    acc_s = term(si_hbm, sj_hbm, 1e-6, False)
    accv[...] = acc_a
    pltpu.sync_copy(accv, out_hbm.at[jnp.int32(0), wid])
    accv[...] = acc_s
    pltpu.sync_copy(accv, out_hbm.at[jnp.int32(1), wid])


@jax.jit
def _sc_call(z, w, ai, aj, si, sj):
    mesh = plsc.VectorSubcoreMesh(core_axis_name="c", subcore_axis_name="s",
                                  num_cores=_NC, num_subcores=_NS)
    f = pl.kernel(
        _sc_body,
        out_type=jax.ShapeDtypeStruct((2, _NW, _LANES), jnp.float32),
        mesh=mesh,
        scratch_types=[
            pltpu.VMEM((2, _CHUNK), jnp.int32),
            pltpu.VMEM((2, _CHUNK), jnp.int32),
            pltpu.VMEM((2, _CHUNK, _D), jnp.float32),
            pltpu.VMEM((2, _CHUNK, _D), jnp.float32),
            pltpu.VMEM((_LANES,), jnp.float32),
            pltpu.SemaphoreType.DMA((2,)),
            pltpu.SemaphoreType.DMA((2,)),
        ],
        compiler_params=pltpu.CompilerParams(needs_layout_passes=False,
                                             use_tc_tiling_on_sc=False),
    )
    return f(z, w, ai, aj, si, sj)


def kernel(latent_z, latent_w, bias, thetas, sparse_i, sparse_j,
           analytical_i, analytical_j):
    z = latent_z.astype(jnp.float32)
    w = latent_w.astype(jnp.float32)
    ai = analytical_i.astype(jnp.int32)
    aj = analytical_j.astype(jnp.int32)
    si = sparse_i.astype(jnp.int32)
    sj = sparse_j.astype(jnp.int32)
    out = _sc_call(z, w, ai, aj, si, sj)
    a_sum = jnp.sum(out[0])
    s_sum = jnp.sum(out[1])
    b = bias[0]
    an_lik = jnp.exp(b - jnp.float32(1e-8)) * a_sum
    return (_E * b - s_sum) - thetas[0] - an_lik


# speedup vs baseline: 1.6646x; 1.6646x over previous
import functools

import jax
import jax.numpy as jnp
from jax import lax
from jax.experimental import pallas as pl
from jax.experimental.pallas import tpu as pltpu
from jax.experimental.pallas import tpu_sc as plsc

_N = 100000
_D = 16
_E = 3200000
_NC = 2
_NS = 16
_NW = _NC * _NS
_LANES = 16
_CHUNK = 1024
_ROWS = _CHUNK // 128
_NCHUNKS = _E // _CHUNK
_K = (_NCHUNKS + _NW - 1) // _NW
_M = _K // 2


def _sqrt16(x):
    i = plsc.bitcast(x, jnp.int32)
    y = plsc.bitcast(jnp.int32(0x5F3759DF) - lax.shift_right_arithmetic(i, jnp.int32(1)),
                     jnp.float32)
    xh = x * jnp.float32(0.5)
    for _ in range(3):
        y = y * (jnp.float32(1.5) - xh * y * y)
    return x * y


def _sc_body(z_hbm, w_hbm, ai_hbm, aj_hbm, si_hbm, sj_hbm, out_hbm,
             ivm, jvm, zr, wr, accv, gsem, isem):
    wid = lax.axis_index("s") * _NC + lax.axis_index("c")
    lanes = lax.iota(jnp.int32, _LANES)

    def term(i_hbm, j_hbm, eps, is_exp):
        def rowbase(k):
            c = jnp.minimum(wid + k * jnp.int32(_NW), jnp.int32(_NCHUNKS - 1))
            return c * jnp.int32(_CHUNK)

        def fetch_idx(k, slot):
            s = jnp.int32(slot)
            r0 = rowbase(k)
            pltpu.async_copy(i_hbm.at[pl.ds(r0, _CHUNK)], ivm.at[s],
                             isem.at[s])
            pltpu.async_copy(j_hbm.at[pl.ds(r0, _CHUNK)], jvm.at[s],
                             isem.at[s])

        def wait_idx(slot):
            s = jnp.int32(slot)
            pltpu.make_async_copy(i_hbm.at[pl.ds(0, _CHUNK)], ivm.at[s],
                                  isem.at[s]).wait()
            pltpu.make_async_copy(j_hbm.at[pl.ds(0, _CHUNK)], jvm.at[s],
                                  isem.at[s]).wait()

        def fire_gathers(slot):
            s = jnp.int32(slot)
            pltpu.async_copy(z_hbm.at[ivm.at[s]], zr.at[s], gsem.at[s])
            pltpu.async_copy(w_hbm.at[jvm.at[s]], wr.at[s], gsem.at[s])

        def wait_gathers(slot):
            s = jnp.int32(slot)
            pltpu.make_async_copy(z_hbm.at[ivm.at[s]], zr.at[s],
                                  gsem.at[s]).wait()
            pltpu.make_async_copy(w_hbm.at[jvm.at[s]], wr.at[s],
                                  gsem.at[s]).wait()

        def compute(k, slot, acc):
            zs, ws = zr.at[jnp.int32(slot)], wr.at[jnp.int32(slot)]

            def group2(g, a):
                base = g * jnp.int32(2 * _LANES)
                sums = []
                for half in range(2):
                    ev = base + jnp.int32(half * _LANES) + lanes
                    s = jnp.zeros((_LANES,), jnp.float32)
                    for d in range(_D):
                        dv = jnp.full((_LANES,), d, jnp.int32)
                        zi = plsc.load_gather(zs, [ev, dv])
                        wj = plsc.load_gather(ws, [ev, dv])
                        t = zi - wj
                        if eps:
                            t = t + jnp.float32(eps)
                        s = s + t * t
                    sums.append(s)
                for s in sums:
                    dist = _sqrt16(s)
                    v = jnp.exp(-dist) if is_exp else dist
                    a = a + v
                return a

            part = lax.fori_loop(jnp.int32(0), jnp.int32(_CHUNK // (2 * _LANES)),
                                 group2, jnp.zeros((_LANES,), jnp.float32))
            valid = (wid + k * jnp.int32(_NW)) < jnp.int32(_NCHUNKS)
            return acc + jnp.where(valid, part, jnp.zeros_like(part))

        fetch_idx(jnp.int32(0), 0)
        wait_idx(0)
        fire_gathers(0)
        fetch_idx(jnp.int32(1), 1)

        def iter2(m, acc):
            k0 = m * jnp.int32(2)
            k1 = k0 + jnp.int32(1)
            last = m >= jnp.int32(_M - 1)
            wait_idx(1)
            fire_gathers(1)

            @pl.when(jnp.logical_not(last))
            def _():
                fetch_idx(k0 + jnp.int32(2), 0)

            wait_gathers(0)
            acc = compute(k0, 0, acc)

            @pl.when(jnp.logical_not(last))
            def _():
                wait_idx(0)
                fire_gathers(0)
                fetch_idx(k1 + jnp.int32(2), 1)

            wait_gathers(1)
            acc = compute(k1, 1, acc)
            return acc

        return lax.fori_loop(jnp.int32(0), jnp.int32(_M), iter2,
                             jnp.zeros((_LANES,), jnp.float32))

    acc_a = term(ai_hbm, aj_hbm, 0.0, True)
    acc_s = term(si_hbm, sj_hbm, 1e-6, False)
    accv[...] = acc_a
    pltpu.sync_copy(accv, out_hbm.at[jnp.int32(0), wid])
    accv[...] = acc_s
    pltpu.sync_copy(accv, out_hbm.at[jnp.int32(1), wid])


@jax.jit
def _sc_call(z, w, ai, aj, si, sj):
    mesh = plsc.VectorSubcoreMesh(core_axis_name="c", subcore_axis_name="s",
                                  num_cores=_NC, num_subcores=_NS)
    f = pl.kernel(
        _sc_body,
        out_type=jax.ShapeDtypeStruct((2, _NW, _LANES), jnp.float32),
        mesh=mesh,
        scratch_types=[
            pltpu.VMEM((2, _CHUNK), jnp.int32),
            pltpu.VMEM((2, _CHUNK), jnp.int32),
            pltpu.VMEM((2, _CHUNK, _D), jnp.float32),
            pltpu.VMEM((2, _CHUNK, _D), jnp.float32),
            pltpu.VMEM((_LANES,), jnp.float32),
            pltpu.SemaphoreType.DMA((2,)),
            pltpu.SemaphoreType.DMA((2,)),
        ],
        compiler_params=pltpu.CompilerParams(needs_layout_passes=False,
                                             use_tc_tiling_on_sc=False),
    )
    return f(z, w, ai, aj, si, sj)


def kernel(latent_z, latent_w, bias, thetas, sparse_i, sparse_j,
           analytical_i, analytical_j):
    z = latent_z.astype(jnp.float32)
    w = latent_w.astype(jnp.float32)
    ai = analytical_i.astype(jnp.int32)
    aj = analytical_j.astype(jnp.int32)
    si = sparse_i.astype(jnp.int32)
    sj = sparse_j.astype(jnp.int32)
    out = _sc_call(z, w, ai, aj, si, sj)
    a_sum = jnp.sum(out[0])
    s_sum = jnp.sum(out[1])
    b = bias[0]
    an_lik = jnp.exp(b - jnp.float32(1e-8)) * a_sum
    return (_E * b - s_sum) - thetas[0] - an_lik


# speedup vs baseline: 2.6200x; 1.5739x over previous
import functools

import jax
import jax.numpy as jnp
from jax import lax
from jax.experimental import pallas as pl
from jax.experimental.pallas import tpu as pltpu
from jax.experimental.pallas import tpu_sc as plsc

_N = 100000
_D = 16
_E = 3200000
_NC = 2
_NS = 16
_NW = _NC * _NS
_LANES = 16
_CHUNK = 1024
_ROWS = _CHUNK // 128
_NCHUNKS = _E // _CHUNK
_K = (_NCHUNKS + _NW - 1) // _NW
_M = _K // 2


def _sqrt16(x):
    i = plsc.bitcast(x, jnp.int32)
    y = plsc.bitcast(jnp.int32(0x5F3759DF) - lax.shift_right_arithmetic(i, jnp.int32(1)),
                     jnp.float32)
    xh = x * jnp.float32(0.5)
    for _ in range(3):
        y = y * (jnp.float32(1.5) - xh * y * y)
    return x * y


def _sc_body(z_hbm, w_hbm, ai_hbm, aj_hbm, si_hbm, sj_hbm, out_hbm,
             ivm, jvm, zr, wr, accv, gsem, isem):
    wid = lax.axis_index("s") * _NC + lax.axis_index("c")
    lanes = lax.iota(jnp.int32, _LANES)

    def term(i_hbm, j_hbm, eps, is_exp):
        def rowbase(k):
            c = jnp.minimum(wid + k * jnp.int32(_NW), jnp.int32(_NCHUNKS - 1))
            return c * jnp.int32(_CHUNK)

        def fetch_idx(k, slot):
            s = jnp.int32(slot)
            r0 = rowbase(k)
            pltpu.async_copy(i_hbm.at[pl.ds(r0, _CHUNK)], ivm.at[s],
                             isem.at[s])
            pltpu.async_copy(j_hbm.at[pl.ds(r0, _CHUNK)], jvm.at[s],
                             isem.at[s])

        def wait_idx(slot):
            s = jnp.int32(slot)
            pltpu.make_async_copy(i_hbm.at[pl.ds(0, _CHUNK)], ivm.at[s],
                                  isem.at[s]).wait()
            pltpu.make_async_copy(j_hbm.at[pl.ds(0, _CHUNK)], jvm.at[s],
                                  isem.at[s]).wait()

        def fire_gathers(slot):
            s = jnp.int32(slot)
            pltpu.async_copy(z_hbm.at[ivm.at[s]], zr.at[s], gsem.at[s])
            pltpu.async_copy(w_hbm.at[jvm.at[s]], wr.at[s], gsem.at[s])

        def wait_gathers(slot):
            s = jnp.int32(slot)
            pltpu.make_async_copy(z_hbm.at[ivm.at[s]], zr.at[s],
                                  gsem.at[s]).wait()
            pltpu.make_async_copy(w_hbm.at[jvm.at[s]], wr.at[s],
                                  gsem.at[s]).wait()

        def compute(k, slot, acc):
            zs, ws = zr.at[jnp.int32(slot)], wr.at[jnp.int32(slot)]

            def group2(g, a):
                base = g * jnp.int32(2 * _LANES)
                sums = []
                for half in range(2):
                    ev = base + jnp.int32(half * _LANES) + lanes
                    s = jnp.zeros((_LANES,), jnp.float32)
                    for d in range(_D):
                        dv = (jnp.full((_LANES,), d, jnp.int32) + lanes) & jnp.int32(_D - 1)
                        zi = plsc.load_gather(zs, [ev, dv])
                        wj = plsc.load_gather(ws, [ev, dv])
                        t = zi - wj
                        if eps:
                            t = t + jnp.float32(eps)
                        s = s + t * t
                    sums.append(s)
                for s in sums:
                    dist = _sqrt16(s)
                    v = jnp.exp(-dist) if is_exp else dist
                    a = a + v
                return a

            part = lax.fori_loop(jnp.int32(0), jnp.int32(_CHUNK // (2 * _LANES)),
                                 group2, jnp.zeros((_LANES,), jnp.float32))
            valid = (wid + k * jnp.int32(_NW)) < jnp.int32(_NCHUNKS)
            return acc + jnp.where(valid, part, jnp.zeros_like(part))

        fetch_idx(jnp.int32(0), 0)
        wait_idx(0)
        fire_gathers(0)
        fetch_idx(jnp.int32(1), 1)

        def iter2(m, acc):
            k0 = m * jnp.int32(2)
            k1 = k0 + jnp.int32(1)
            last = m >= jnp.int32(_M - 1)
            wait_idx(1)
            fire_gathers(1)

            @pl.when(jnp.logical_not(last))
            def _():
                fetch_idx(k0 + jnp.int32(2), 0)

            wait_gathers(0)
            acc = compute(k0, 0, acc)

            @pl.when(jnp.logical_not(last))
            def _():
                wait_idx(0)
                fire_gathers(0)
                fetch_idx(k1 + jnp.int32(2), 1)

            wait_gathers(1)
            acc = compute(k1, 1, acc)
            return acc

        return lax.fori_loop(jnp.int32(0), jnp.int32(_M), iter2,
                             jnp.zeros((_LANES,), jnp.float32))

    acc_a = term(ai_hbm, aj_hbm, 0.0, True)
    acc_s = term(si_hbm, sj_hbm, 1e-6, False)
    accv[...] = acc_a
    pltpu.sync_copy(accv, out_hbm.at[jnp.int32(0), wid])
    accv[...] = acc_s
    pltpu.sync_copy(accv, out_hbm.at[jnp.int32(1), wid])


@jax.jit
def _sc_call(z, w, ai, aj, si, sj):
    mesh = plsc.VectorSubcoreMesh(core_axis_name="c", subcore_axis_name="s",
                                  num_cores=_NC, num_subcores=_NS)
    f = pl.kernel(
        _sc_body,
        out_type=jax.ShapeDtypeStruct((2, _NW, _LANES), jnp.float32),
        mesh=mesh,
        scratch_types=[
            pltpu.VMEM((2, _CHUNK), jnp.int32),
            pltpu.VMEM((2, _CHUNK), jnp.int32),
            pltpu.VMEM((2, _CHUNK, _D), jnp.float32),
            pltpu.VMEM((2, _CHUNK, _D), jnp.float32),
            pltpu.VMEM((_LANES,), jnp.float32),
            pltpu.SemaphoreType.DMA((2,)),
            pltpu.SemaphoreType.DMA((2,)),
        ],
        compiler_params=pltpu.CompilerParams(needs_layout_passes=False,
                                             use_tc_tiling_on_sc=False),
    )
    return f(z, w, ai, aj, si, sj)


def kernel(latent_z, latent_w, bias, thetas, sparse_i, sparse_j,
           analytical_i, analytical_j):
    z = latent_z.astype(jnp.float32)
    w = latent_w.astype(jnp.float32)
    ai = analytical_i.astype(jnp.int32)
    aj = analytical_j.astype(jnp.int32)
    si = sparse_i.astype(jnp.int32)
    sj = sparse_j.astype(jnp.int32)
    out = _sc_call(z, w, ai, aj, si, sj)
    a_sum = jnp.sum(out[0])
    s_sum = jnp.sum(out[1])
    b = bias[0]
    an_lik = jnp.exp(b - jnp.float32(1e-8)) * a_sum
    return (_E * b - s_sum) - thetas[0] - an_lik


# speedup vs baseline: 3.2423x; 1.2375x over previous
import functools

import jax
import jax.numpy as jnp
from jax import lax
from jax.experimental import pallas as pl
from jax.experimental.pallas import tpu as pltpu
from jax.experimental.pallas import tpu_sc as plsc

_N = 100000
_D = 16
_E = 3200000
_NC = 2
_NS = 16
_NW = _NC * _NS
_LANES = 16
_CHUNK = 1024
_NCHUNKS = _E // _CHUNK
_K = (_NCHUNKS + _NW - 1) // _NW
_M = _K // 2


def _sqrt16(x):
    i = plsc.bitcast(x, jnp.int32)
    y = plsc.bitcast(jnp.int32(0x5F3759DF) - lax.shift_right_arithmetic(i, jnp.int32(1)),
                     jnp.float32)
    xh = x * jnp.float32(0.5)
    for _ in range(3):
        y = y * (jnp.float32(1.5) - xh * y * y)
    return x * y


def _make_body(eps, is_exp):
    def body(z_hbm, w_hbm, i_hbm, j_hbm, out_hbm,
             ivm, jvm, zr, wr, accv, gsem, isem):
        wid = lax.axis_index("s") * _NC + lax.axis_index("c")
        lanes = lax.iota(jnp.int32, _LANES)

        def rowbase(k):
            c = jnp.minimum(wid + k * jnp.int32(_NW), jnp.int32(_NCHUNKS - 1))
            return c * jnp.int32(_CHUNK)

        def fetch_idx(k, slot):
            s = jnp.int32(slot)
            r0 = rowbase(k)
            pltpu.async_copy(i_hbm.at[pl.ds(r0, _CHUNK)], ivm.at[s],
                             isem.at[s])
            pltpu.async_copy(j_hbm.at[pl.ds(r0, _CHUNK)], jvm.at[s],
                             isem.at[s])

        def wait_idx(slot):
            s = jnp.int32(slot)
            pltpu.make_async_copy(i_hbm.at[pl.ds(0, _CHUNK)], ivm.at[s],
                                  isem.at[s]).wait()
            pltpu.make_async_copy(j_hbm.at[pl.ds(0, _CHUNK)], jvm.at[s],
                                  isem.at[s]).wait()

        def fire_gathers(slot):
            s = jnp.int32(slot)
            pltpu.async_copy(z_hbm.at[ivm.at[s]], zr.at[s], gsem.at[s])
            pltpu.async_copy(w_hbm.at[jvm.at[s]], wr.at[s], gsem.at[s])

        def wait_gathers(slot):
            s = jnp.int32(slot)
            pltpu.make_async_copy(z_hbm.at[ivm.at[s]], zr.at[s],
                                  gsem.at[s]).wait()
            pltpu.make_async_copy(w_hbm.at[jvm.at[s]], wr.at[s],
                                  gsem.at[s]).wait()

        def compute(k, slot, acc):
            zs, ws = zr.at[jnp.int32(slot)], wr.at[jnp.int32(slot)]

            def group2(g, a):
                base = g * jnp.int32(2 * _LANES)
                sums = []
                for half in range(2):
                    ev = base + jnp.int32(half * _LANES) + lanes
                    s = jnp.zeros((_LANES,), jnp.float32)
                    for d in range(_D):
                        dv = (jnp.full((_LANES,), d, jnp.int32) + lanes) & jnp.int32(_D - 1)
                        zi = plsc.load_gather(zs, [ev, dv])
                        wj = plsc.load_gather(ws, [ev, dv])
                        t = zi - wj
                        if eps:
                            t = t + jnp.float32(eps)
                        s = s + t * t
                    sums.append(s)
                for s in sums:
                    dist = _sqrt16(s)
                    v = jnp.exp(-dist) if is_exp else dist
                    a = a + v
                return a

            part = lax.fori_loop(jnp.int32(0), jnp.int32(_CHUNK // (2 * _LANES)),
                                 group2, jnp.zeros((_LANES,), jnp.float32))
            valid = (wid + k * jnp.int32(_NW)) < jnp.int32(_NCHUNKS)
            return acc + jnp.where(valid, part, jnp.zeros_like(part))

        fetch_idx(jnp.int32(0), 0)
        wait_idx(0)
        fire_gathers(0)
        fetch_idx(jnp.int32(1), 1)

        def iter2(m, acc):
            k0 = m * jnp.int32(2)
            k1 = k0 + jnp.int32(1)
            last = m >= jnp.int32(_M - 1)
            wait_idx(1)
            fire_gathers(1)

            @pl.when(jnp.logical_not(last))
            def _():
                fetch_idx(k0 + jnp.int32(2), 0)

            wait_gathers(0)
            acc = compute(k0, 0, acc)

            @pl.when(jnp.logical_not(last))
            def _():
                wait_idx(0)
                fire_gathers(0)
                fetch_idx(k1 + jnp.int32(2), 1)

            wait_gathers(1)
            acc = compute(k1, 1, acc)
            return acc

        acc = lax.fori_loop(jnp.int32(0), jnp.int32(_M), iter2,
                            jnp.zeros((_LANES,), jnp.float32))
        accv[...] = acc
        pltpu.sync_copy(accv, out_hbm.at[wid])

    return body


def _term_call(z, w, i32, j32, eps, is_exp):
    mesh = plsc.VectorSubcoreMesh(core_axis_name="c", subcore_axis_name="s",
                                  num_cores=_NC, num_subcores=_NS)
    f = pl.kernel(
        _make_body(eps, is_exp),
        out_type=jax.ShapeDtypeStruct((_NW, _LANES), jnp.float32),
        mesh=mesh,
        scratch_types=[
            pltpu.VMEM((2, _CHUNK), jnp.int32),
            pltpu.VMEM((2, _CHUNK), jnp.int32),
            pltpu.VMEM((2, _CHUNK, _D), jnp.float32),
            pltpu.VMEM((2, _CHUNK, _D), jnp.float32),
            pltpu.VMEM((_LANES,), jnp.float32),
            pltpu.SemaphoreType.DMA((2,)),
            pltpu.SemaphoreType.DMA((2,)),
        ],
        compiler_params=pltpu.CompilerParams(needs_layout_passes=False,
                                             use_tc_tiling_on_sc=False),
    )
    return f(z, w, i32, j32)


def kernel(latent_z, latent_w, bias, thetas, sparse_i, sparse_j,
           analytical_i, analytical_j):
    z = latent_z.astype(jnp.float32)
    w = latent_w.astype(jnp.float32)
    ai = analytical_i.astype(jnp.int32)
    aj = analytical_j.astype(jnp.int32)
    out_a = _term_call(z, w, ai, aj, 0.0, True)
    si = sparse_i.astype(jnp.int32)
    sj = sparse_j.astype(jnp.int32)
    out_s = _term_call(z, w, si, sj, 1e-6, False)
    a_sum = jnp.sum(out_a)
    s_sum = jnp.sum(out_s)
    b = bias[0]
    an_lik = jnp.exp(b - jnp.float32(1e-8)) * a_sum
    return (_E * b - s_sum) - thetas[0] - an_lik
